# baseline (speedup 1.0000x reference)
import functools

import jax
import jax.numpy as jnp
from jax import lax
from jax.experimental import pallas as pl
from jax.experimental.pallas import tpu as pltpu
from jax.experimental.pallas import tpu_sc as plsc

_NC = 2
_NS = 16
_LANES = 16
_BBLK = 128
_NBUF = 5
_LEAD = 4
_OPAD = _BBLK + 1
_SUB = 8


def kernel(input_ids, word_embeddings, position_embeddings):
    B, L = input_ids.shape
    V, H = word_embeddings.shape
    NW = _NC * _NS
    assert B == NW * _BBLK
    assert L % _NBUF == 0
    nq = H // _LANES
    nt = H // _SUB

    ids_t = input_ids.T
    pos_t = position_embeddings.T

    mesh = plsc.VectorSubcoreMesh(core_axis_name="c", subcore_axis_name="s")

    @functools.partial(
        pl.kernel,
        out_type=jax.ShapeDtypeStruct((L, nt, NW, _SUB, _BBLK), jnp.float32),
        mesh=mesh,
        scratch_types=[
            pltpu.VMEM((L, _BBLK), jnp.int32),
            pltpu.VMEM((H, 256), jnp.float32),
            [pltpu.VMEM((_BBLK, H), jnp.float32) for _ in range(_NBUF)],
            [pltpu.VMEM((nt, _SUB, _OPAD), jnp.float32) for _ in range(_NBUF)],
            [pltpu.SemaphoreType.DMA for _ in range(_NBUF)],
            [pltpu.SemaphoreType.DMA for _ in range(_NBUF)],
        ],
        compiler_params=pltpu.CompilerParams(
            needs_layout_passes=False, use_tc_tiling_on_sc=False),
    )
    def emb_kernel(ids_hbm, tab_hbm, pos_hbm, out_hbm, idx_v, pos_v,
                   gbufs, obufs, gsems, osems):
        wid = lax.axis_index("s") * _NC + lax.axis_index("c")
        b0 = wid * _BBLK
        pltpu.sync_copy(ids_hbm.at[:, pl.ds(b0, _BBLK)], idx_v)
        pltpu.sync_copy(pos_hbm.at[:, pl.ds(0, 256)], pos_v)

        def start_gather(l, slot):
            pltpu.async_copy(tab_hbm.at[idx_v.at[l]], gbufs[slot], gsems[slot])

        def wait_gather(slot):
            pltpu.make_async_copy(tab_hbm.at[idx_v.at[0]], gbufs[slot],
                                  gsems[slot]).wait()

        def start_out(l, slot):
            pltpu.async_copy(obufs[slot].at[:, :, pl.ds(0, _BBLK)],
                             out_hbm.at[l, :, wid],
                             osems[slot])

        def wait_out(slot):
            pltpu.make_async_copy(obufs[slot].at[:, :, pl.ds(0, _BBLK)],
                                  out_hbm.at[0, :, 0],
                                  osems[slot]).wait()

        for s in range(_LEAD):
            start_gather(s, s)

        hq = [lax.iota(jnp.int32, _LANES) + q * _LANES for q in range(nq)]
        tq = [h // _SUB for h in hq]
        sq = [lax.rem(h, _SUB) for h in hq]

        def compute(l, slot):
            gbuf, obuf = gbufs[slot], obufs[slot]

            lvec = jnp.full((_LANES,), l, dtype=jnp.int32)
            pos_col = [plsc.load_gather(pos_v, [hq[q], lvec])
                       for q in range(nq)]

            @plsc.parallel_loop(0, _BBLK, unroll=8)
            def j_body(j):
                jvec = jnp.full((_LANES,), j, dtype=jnp.int32)
                for q in range(nq):
                    v = gbuf[j, pl.ds(q * _LANES, _LANES)] + pos_col[q]
                    plsc.store_scatter(obuf, [tq[q], sq[q], jvec], v)

        @pl.loop(0, L // _NBUF)
        def _round(r):
            for s in range(_NBUF):
                l = r * _NBUF + s
                wait_gather(s)

                @pl.when(l >= _NBUF)
                def _():
                    wait_out(s)
                compute(l, s)
                start_out(l, s)
                sa = (s + _LEAD) % _NBUF

                @pl.when(l + _LEAD < L)
                def _():
                    start_gather(l + _LEAD, sa)

        for s in range(_NBUF):
            wait_out(s)

    out5 = emb_kernel(ids_t, word_embeddings, pos_t)
    out = out5.transpose(2, 4, 0, 1, 3).reshape(B, L, H)
    return out


# speedup vs baseline: 1.0039x; 1.0039x over previous
import functools

import jax
import jax.numpy as jnp
from jax import lax
from jax.experimental import pallas as pl
from jax.experimental.pallas import tpu as pltpu
from jax.experimental.pallas import tpu_sc as plsc

_NC = 2
_NS = 16
_LANES = 16
_BBLK = 128
_NBUF = 5
_LEAD = 3
_OPAD = _BBLK + 1
_SUB = 8


def kernel(input_ids, word_embeddings, position_embeddings):
    B, L = input_ids.shape
    V, H = word_embeddings.shape
    NW = _NC * _NS
    assert B == NW * _BBLK
    assert L % _NBUF == 0
    nq = H // _LANES
    nt = H // _SUB

    ids_t = input_ids.T
    pos_t = position_embeddings.T

    mesh = plsc.VectorSubcoreMesh(core_axis_name="c", subcore_axis_name="s")

    @functools.partial(
        pl.kernel,
        out_type=jax.ShapeDtypeStruct((L, nt, NW, _SUB, _BBLK), jnp.float32),
        mesh=mesh,
        scratch_types=[
            pltpu.VMEM((L, _BBLK), jnp.int32),
            pltpu.VMEM((H, 256), jnp.float32),
            [pltpu.VMEM((_BBLK, H), jnp.float32) for _ in range(_NBUF)],
            [pltpu.VMEM((nt, _SUB, _OPAD), jnp.float32) for _ in range(_NBUF)],
            [pltpu.SemaphoreType.DMA for _ in range(_NBUF)],
            [pltpu.SemaphoreType.DMA for _ in range(_NBUF)],
        ],
        compiler_params=pltpu.CompilerParams(
            needs_layout_passes=False, use_tc_tiling_on_sc=False),
    )
    def emb_kernel(ids_hbm, tab_hbm, pos_hbm, out_hbm, idx_v, pos_v,
                   gbufs, obufs, gsems, osems):
        wid = lax.axis_index("s") * _NC + lax.axis_index("c")
        b0 = wid * _BBLK
        pltpu.sync_copy(ids_hbm.at[:, pl.ds(b0, _BBLK)], idx_v)
        pltpu.sync_copy(pos_hbm.at[:, pl.ds(0, 256)], pos_v)

        def start_gather(l, slot):
            pltpu.async_copy(tab_hbm.at[idx_v.at[l]], gbufs[slot], gsems[slot])

        def wait_gather(slot):
            pltpu.make_async_copy(tab_hbm.at[idx_v.at[0]], gbufs[slot],
                                  gsems[slot]).wait()

        def start_out(l, slot):
            pltpu.async_copy(obufs[slot].at[:, :, pl.ds(0, _BBLK)],
                             out_hbm.at[l, :, wid],
                             osems[slot])

        def wait_out(slot):
            pltpu.make_async_copy(obufs[slot].at[:, :, pl.ds(0, _BBLK)],
                                  out_hbm.at[0, :, 0],
                                  osems[slot]).wait()

        for s in range(_LEAD):
            start_gather(s, s)

        hq = [lax.iota(jnp.int32, _LANES) + q * _LANES for q in range(nq)]
        tq = [h // _SUB for h in hq]
        sq = [lax.rem(h, _SUB) for h in hq]

        def compute(l, slot):
            gbuf, obuf = gbufs[slot], obufs[slot]

            lvec = jnp.full((_LANES,), l, dtype=jnp.int32)
            pos_col = [plsc.load_gather(pos_v, [hq[q], lvec])
                       for q in range(nq)]

            @plsc.parallel_loop(0, _BBLK, unroll=8)
            def j_body(j):
                jvec = jnp.full((_LANES,), j, dtype=jnp.int32)
                for q in range(nq):
                    v = gbuf[j, pl.ds(q * _LANES, _LANES)] + pos_col[q]
                    plsc.store_scatter(obuf, [tq[q], sq[q], jvec], v)

        @pl.loop(0, L // _NBUF)
        def _round(r):
            for s in range(_NBUF):
                l = r * _NBUF + s
                wait_gather(s)

                @pl.when(l >= _NBUF)
                def _():
                    wait_out(s)
                compute(l, s)
                start_out(l, s)
                sa = (s + _LEAD) % _NBUF

                @pl.when(l + _LEAD < L)
                def _():
                    start_gather(l + _LEAD, sa)

        for s in range(_NBUF):
            wait_out(s)

    out5 = emb_kernel(ids_t, word_embeddings, pos_t)
    out = out5.transpose(2, 4, 0, 1, 3).reshape(B, L, H)
    return out
